# Initial kernel scaffold; baseline (speedup 1.0000x reference)
import jax
import jax.numpy as jnp
from jax.experimental import pallas as pl


def kernel(node_ids, edge_index, embed_table, weight, bias, W_att, v_att):
    raise NotImplementedError("write your pallas kernel here")


# speedup vs baseline: 2.3442x; 2.3442x over previous
import functools

import jax
import jax.numpy as jnp
from jax import lax
from jax.experimental import pallas as pl
from jax.experimental.pallas import tpu as pltpu
from jax.experimental.pallas import tpu_sc as plsc

N_NODES = 10000
EMBED = 128
E = 320000
B = 1024
L = 50

NC = 2
NS = 16
NW = NC * NS

NP = 10112
ROWS_PER_TILE = NP // NS
CH = 128
E_PAD = 323584
CPW = E_PAD // (NW * CH)

L_PAD = 64
B_IDX = B * L_PAD
IDS_PER_W = B_IDX // NW
GCH = IDS_PER_W // CH

DEGW = 128

_mesh = plsc.VectorSubcoreMesh(
    core_axis_name="c", subcore_axis_name="s", num_cores=NC, num_subcores=NS)


def _zero_vmem(buf, rows, cols):
  z = jnp.zeros((16,), jnp.float32)

  def body(i, _):
    for j in range(cols // 16):
      buf[i, pl.ds(j * 16, 16)] = z
    return 0

  lax.fori_loop(0, rows, body, 0)


def _fill_ones(buf, rows, cols):
  o = jnp.ones((16,), jnp.float32)

  def body(i, _):
    for j in range(cols // 16):
      buf[i, pl.ds(j * 16, 16)] = o
    return 0

  lax.fori_loop(0, rows, body, 0)


def _hop_body(h_hbm, src_hbm, dst_hbm, p_out, idx_s, idx_d, rows, zbuf, acc,
              sem):
  c = lax.axis_index("c")
  s = lax.axis_index("s")
  base = s * ROWS_PER_TILE

  _zero_vmem(zbuf, 64, EMBED)
  for k in range(9):
    pltpu.sync_copy(zbuf, acc.at[pl.ds(base + k * 64, 64)])
  pltpu.sync_copy(zbuf.at[pl.ds(0, 56)], acc.at[pl.ds(base + 576, 56)])
  plsc.subcore_barrier()

  wid = s * NC + c
  ebase = wid * (CPW * CH)

  def body(j, _):
    off = ebase + j * CH
    pltpu.sync_copy(src_hbm.at[pl.ds(off, CH)], idx_s)
    pltpu.sync_copy(dst_hbm.at[pl.ds(off, CH)], idx_d)
    pltpu.async_copy(h_hbm.at[idx_s], rows, sem).wait()
    pltpu.sync_copy(rows, acc.at[idx_d], add=True)
    return 0

  lax.fori_loop(0, CPW, body, 0)
  plsc.subcore_barrier()

  pltpu.sync_copy(acc.at[pl.ds(base, ROWS_PER_TILE)],
                  p_out.at[c, pl.ds(base, ROWS_PER_TILE)])


_hop = pl.kernel(
    _hop_body,
    out_type=jax.ShapeDtypeStruct((NC, NP, EMBED), jnp.float32),
    mesh=_mesh,
    scratch_types=[
        pltpu.VMEM((CH,), jnp.int32),
        pltpu.VMEM((CH,), jnp.int32),
        pltpu.VMEM((CH, EMBED), jnp.float32),
        pltpu.VMEM((64, EMBED), jnp.float32),
        pltpu.VMEM_SHARED((NP, EMBED), jnp.float32),
        pltpu.SemaphoreType.DMA,
    ],
)


def _deg_body(dst_hbm, degp_out, idx_d, ones, zdeg, degacc, sem):
  del sem
  c = lax.axis_index("c")
  s = lax.axis_index("s")
  base = s * ROWS_PER_TILE

  _fill_ones(ones, CH, DEGW)
  _zero_vmem(zdeg, 64, DEGW)
  for k in range(9):
    pltpu.sync_copy(zdeg, degacc.at[pl.ds(base + k * 64, 64)])
  pltpu.sync_copy(zdeg.at[pl.ds(0, 56)], degacc.at[pl.ds(base + 576, 56)])
  plsc.subcore_barrier()

  wid = s * NC + c
  ebase = wid * (CPW * CH)

  def body(j, _):
    off = ebase + j * CH
    pltpu.sync_copy(dst_hbm.at[pl.ds(off, CH)], idx_d)
    pltpu.sync_copy(ones, degacc.at[idx_d], add=True)
    return 0

  lax.fori_loop(0, CPW, body, 0)
  plsc.subcore_barrier()

  pltpu.sync_copy(degacc.at[pl.ds(base, ROWS_PER_TILE)],
                  degp_out.at[c, pl.ds(base, ROWS_PER_TILE)])


_deg = pl.kernel(
    _deg_body,
    out_type=jax.ShapeDtypeStruct((NC, NP, DEGW), jnp.float32),
    mesh=_mesh,
    scratch_types=[
        pltpu.VMEM((CH,), jnp.int32),
        pltpu.VMEM((CH, DEGW), jnp.float32),
        pltpu.VMEM((64, DEGW), jnp.float32),
        pltpu.VMEM_SHARED((NP, DEGW), jnp.float32),
        pltpu.SemaphoreType.DMA,
    ],
)


def _gather_body(tab_hbm, ids_hbm, out_hbm, idx, rows, sem):
  c = lax.axis_index("c")
  s = lax.axis_index("s")
  wid = s * NC + c
  base = wid * IDS_PER_W

  def body(j, _):
    off = base + j * CH
    pltpu.sync_copy(ids_hbm.at[pl.ds(off, CH)], idx)
    pltpu.async_copy(tab_hbm.at[idx], rows, sem).wait()
    pltpu.sync_copy(rows, out_hbm.at[pl.ds(off, CH)])
    return 0

  lax.fori_loop(0, GCH, body, 0)


_gather = pl.kernel(
    _gather_body,
    out_type=jax.ShapeDtypeStruct((B_IDX, EMBED), jnp.float32),
    mesh=_mesh,
    scratch_types=[
        pltpu.VMEM((CH,), jnp.int32),
        pltpu.VMEM((CH, EMBED), jnp.float32),
        pltpu.SemaphoreType.DMA,
    ],
)


_CB = 632


def _combine_body(add_raw, p_ref, dp_ref, *rest):
  if add_raw:
    raw_ref, out_ref = rest
  else:
    (out_ref,) = rest
  acc = p_ref[0] + p_ref[1]
  deg = dp_ref[0, :, 0:1] + dp_ref[1, :, 0:1]
  deg = jnp.maximum(deg, 1.0)
  h = acc / deg
  if add_raw:
    h = h + raw_ref[...]
  out_ref[...] = h


def _make_combine(add_raw):
  in_specs = [
      pl.BlockSpec((NC, _CB, EMBED), lambda i: (0, i, 0)),
      pl.BlockSpec((NC, _CB, DEGW), lambda i: (0, i, 0)),
  ]
  if add_raw:
    in_specs.append(pl.BlockSpec((_CB, EMBED), lambda i: (i, 0)))
  return pl.pallas_call(
      functools.partial(_combine_body, add_raw),
      grid=(NP // _CB,),
      in_specs=in_specs,
      out_specs=pl.BlockSpec((_CB, EMBED), lambda i: (i, 0)),
      out_shape=jax.ShapeDtypeStruct((NP, EMBED), jnp.float32),
  )


_BB = 16
_R = _BB * L_PAD


def _dense_body(g_ref, w_ref, b_ref, wa_ref, v_ref, out_ref):
  x = g_ref[...]
  feat = jnp.maximum(
      jnp.dot(x, w_ref[...], precision=lax.Precision.HIGHEST) + b_ref[...],
      0.0)
  t = jnp.tanh(jnp.dot(feat, wa_ref[...], precision=lax.Precision.HIGHEST))
  sc = jnp.sum(t * v_ref[...], axis=1, keepdims=True)
  r = lax.broadcasted_iota(jnp.int32, (_R, 1), 0)
  valid = ((r % L_PAD) < L).astype(jnp.float32)
  e = jnp.exp(sc) * valid
  rb = lax.broadcasted_iota(jnp.int32, (_BB, _R), 0)
  rr = lax.broadcasted_iota(jnp.int32, (_BB, _R), 1)
  sel = (rr // L_PAD == rb).astype(jnp.float32)
  num = jnp.dot(sel, feat * e, precision=lax.Precision.HIGHEST)
  den = jnp.dot(sel, e, precision=lax.Precision.HIGHEST)
  out_ref[...] = num / den


_dense = pl.pallas_call(
    _dense_body,
    grid=(B // _BB,),
    in_specs=[
        pl.BlockSpec((_R, EMBED), lambda i: (i, 0)),
        pl.BlockSpec((EMBED, EMBED), lambda i: (0, 0)),
        pl.BlockSpec((1, EMBED), lambda i: (0, 0)),
        pl.BlockSpec((EMBED, EMBED), lambda i: (0, 0)),
        pl.BlockSpec((1, EMBED), lambda i: (0, 0)),
    ],
    out_specs=pl.BlockSpec((_BB, EMBED), lambda i: (i, 0)),
    out_shape=jax.ShapeDtypeStruct((B, EMBED), jnp.float32),
)


def kernel(node_ids, edge_index, embed_table, weight, bias, W_att, v_att):
  src = edge_index[0]
  dst = edge_index[1]
  pad_e = E_PAD - E
  src_p = jnp.concatenate([src, jnp.zeros((pad_e,), jnp.int32)])
  dst_p = jnp.concatenate([dst, jnp.full((pad_e,), NP - 1, jnp.int32)])
  embed_pad = jnp.pad(embed_table, ((0, NP - N_NODES), (0, 0)))

  comb1 = _make_combine(False)
  comb2 = _make_combine(True)

  degp = _deg(dst_p)
  p1 = _hop(embed_pad, src_p, dst_p)
  h1 = comb1(p1, degp)
  p2 = _hop(h1, src_p, dst_p)
  hsum = comb2(p2, degp, embed_pad)

  ids = jnp.pad(node_ids, ((0, 0), (0, L_PAD - L))).reshape(-1)
  g = _gather(hsum, ids)
  out = _dense(g, weight, bias.reshape(1, EMBED), W_att,
               v_att.reshape(1, EMBED))
  return out
